# Initial kernel scaffold; baseline (speedup 1.0000x reference)
import jax
import jax.numpy as jnp
from jax.experimental import pallas as pl


def kernel(x, edge_index, edge_attr, edge_type, W_nenc, b_nenc, W_eenc, b_eenc, W1_rel, W1_root, W1_edge, b1, W2_rel, W2_root, b2, W_v, b_v, W_a, b_a):
    raise NotImplementedError("write your pallas kernel here")


# speedup vs baseline: 1.3190x; 1.3190x over previous
import jax
import jax.numpy as jnp
from jax.experimental import pallas as pl

_HID = 64
_EHID = 32
_NODE_BLK = 2000
_EDGE_BLK = 8000


def _stage1_node_kernel(x_ref, wn_ref, bn_ref, w1rel_ref, w1root_ref,
                        n_ref, xw0_ref, xw1_ref, root_ref):
    x = x_ref[...]
    wn = wn_ref[...]
    acc = bn_ref[...]
    n = (x[:, 0:1] * wn[0:1, :] + x[:, 1:2] * wn[1:2, :]
         + x[:, 2:3] * wn[2:3, :] + acc)
    n = jnp.maximum(n, 0.0)
    n_ref[...] = n
    w1rel = w1rel_ref[...]
    xw0_ref[...] = jnp.dot(n, w1rel[0], preferred_element_type=jnp.float32)
    xw1_ref[...] = jnp.dot(n, w1rel[1], preferred_element_type=jnp.float32)
    root_ref[...] = jnp.dot(n, w1root_ref[...],
                            preferred_element_type=jnp.float32)


def _edge_kernel(ea_ref, we_ref, be_ref, w1e_ref, ew_ref):
    ea = ea_ref[...]
    we = we_ref[...]
    e = ea[:, 0:1] * we[0:1, :] + ea[:, 1:2] * we[1:2, :] + be_ref[...]
    e = jnp.maximum(e, 0.0)
    ew_ref[...] = jnp.dot(e, w1e_ref[...], preferred_element_type=jnp.float32)


def _stage2_node_kernel(agg_ref, root_ref, b1_ref, w2rel_ref, w2root_ref,
                        b2_ref, hw0_ref, hw1_ref, outroot_ref):
    h = jnp.maximum(agg_ref[...] + root_ref[...] + b1_ref[...], 0.0)
    w2rel = w2rel_ref[...]
    hw0_ref[...] = jnp.dot(h, w2rel[0], preferred_element_type=jnp.float32)
    hw1_ref[...] = jnp.dot(h, w2rel[1], preferred_element_type=jnp.float32)
    outroot_ref[...] = (jnp.dot(h, w2root_ref[...],
                                preferred_element_type=jnp.float32)
                        + b2_ref[...])


def _head_kernel(outroot_ref, asum_ref, wva_ref, bva_ref, va_ref):
    h2 = jnp.maximum(outroot_ref[...] + asum_ref[...], 0.0)
    va_ref[...] = (jnp.dot(h2, wva_ref[...],
                           preferred_element_type=jnp.float32) + bva_ref[...])


def kernel(x, edge_index, edge_attr, edge_type, W_nenc, b_nenc, W_eenc,
           b_eenc, W1_rel, W1_root, W1_edge, b1, W2_rel, W2_root, b2,
           W_v, b_v, W_a, b_a):
    num_nodes = x.shape[0]
    num_edges = edge_attr.shape[0]
    src = edge_index[0]
    dst = edge_index[1]

    nb = num_nodes // _NODE_BLK
    eb = num_edges // _EDGE_BLK

    full2 = lambda i: (0, 0)
    full3 = lambda i: (0, 0, 0)
    rowblk = lambda i: (i, 0)

    n, xw0, xw1, root = pl.pallas_call(
        _stage1_node_kernel,
        grid=(nb,),
        in_specs=[
            pl.BlockSpec((_NODE_BLK, 3), rowblk),
            pl.BlockSpec((3, _HID), full2),
            pl.BlockSpec((1, _HID), full2),
            pl.BlockSpec((2, _HID, _HID), full3),
            pl.BlockSpec((_HID, _HID), full2),
        ],
        out_specs=[pl.BlockSpec((_NODE_BLK, _HID), rowblk)] * 4,
        out_shape=[jax.ShapeDtypeStruct((num_nodes, _HID), jnp.float32)] * 4,
    )(x, W_nenc, b_nenc.reshape(1, _HID), W1_rel, W1_root)

    ew = pl.pallas_call(
        _edge_kernel,
        grid=(eb,),
        in_specs=[
            pl.BlockSpec((_EDGE_BLK, 2), rowblk),
            pl.BlockSpec((2, _EHID), full2),
            pl.BlockSpec((1, _EHID), full2),
            pl.BlockSpec((_EHID, _HID), full2),
        ],
        out_specs=pl.BlockSpec((_EDGE_BLK, _HID), rowblk),
        out_shape=jax.ShapeDtypeStruct((num_edges, _HID), jnp.float32),
    )(edge_attr, W_eenc, b_eenc.reshape(1, _EHID), W1_edge)

    is0 = (edge_type == 0)[:, None]
    msg = jnp.where(is0, xw0[src], xw1[src]) + ew
    agg = jax.ops.segment_sum(msg, dst, num_segments=num_nodes)

    hw0, hw1, outroot = pl.pallas_call(
        _stage2_node_kernel,
        grid=(nb,),
        in_specs=[
            pl.BlockSpec((_NODE_BLK, _HID), rowblk),
            pl.BlockSpec((_NODE_BLK, _HID), rowblk),
            pl.BlockSpec((1, _HID), full2),
            pl.BlockSpec((2, _HID, _HID), full3),
            pl.BlockSpec((_HID, _HID), full2),
            pl.BlockSpec((1, _HID), full2),
        ],
        out_specs=[pl.BlockSpec((_NODE_BLK, _HID), rowblk)] * 3,
        out_shape=[jax.ShapeDtypeStruct((num_nodes, _HID), jnp.float32)] * 3,
    )(agg, root, b1.reshape(1, _HID), W2_rel, W2_root, b2.reshape(1, _HID))

    neg = jnp.float32(-jnp.inf)
    m0 = jnp.where(is0, hw0[src], neg)
    m1 = jnp.where(is0, neg, hw1[src])
    a0 = jax.ops.segment_max(m0, dst, num_segments=num_nodes)
    a1 = jax.ops.segment_max(m1, dst, num_segments=num_nodes)
    a0 = jnp.where(jnp.isfinite(a0), a0, 0.0)
    a1 = jnp.where(jnp.isfinite(a1), a1, 0.0)
    asum = a0 + a1

    w_va = jnp.zeros((_HID, 128), jnp.float32)
    w_va = w_va.at[:, 0:1].set(W_v).at[:, 1:6].set(W_a)
    b_va = jnp.zeros((1, 128), jnp.float32)
    b_va = b_va.at[0, 0].set(b_v[0]).at[0, 1:6].set(b_a)

    va = pl.pallas_call(
        _head_kernel,
        grid=(nb,),
        in_specs=[
            pl.BlockSpec((_NODE_BLK, _HID), rowblk),
            pl.BlockSpec((_NODE_BLK, _HID), rowblk),
            pl.BlockSpec((_HID, 128), full2),
            pl.BlockSpec((1, 128), full2),
        ],
        out_specs=pl.BlockSpec((_NODE_BLK, 128), rowblk),
        out_shape=jax.ShapeDtypeStruct((num_nodes, 128), jnp.float32),
    )(outroot, asum, w_va, b_va)

    value = va[:, 0:1]
    action = va[:, 1:6]
    return (value, action)
